# baseline (speedup 1.0000x reference)
import functools

import jax
import jax.numpy as jnp
from jax import lax
from jax.experimental import pallas as pl
from jax.experimental.pallas import tpu as pltpu
from jax.experimental.pallas import tpu_sc as plsc

B, Din, Dp, K, N, Dout = 4, 32, 3, 16, 16384, 32
BN = B * N
DROW = (1 + Dp) * Din

NC, NS = 2, 16
NW = NC * NS
NHALF = 2
BNH = BN // NHALF
QPW = BNH // NW
CP = 8
CPK = CP * K
NCHUNK = QPW // CP
RING = 8

_sc_mesh = plsc.VectorSubcoreMesh(core_axis_name="c", subcore_axis_name="s")

_gdn = lax.GatherDimensionNumbers(
    offset_dims=(), collapsed_slice_dims=(0,), start_index_map=(0,)
)


def _bcast_lane(v, k):
    idx = jnp.full((K, 1), k, jnp.int32)
    return lax.gather(v, idx, _gdn, slice_sizes=(1,),
                      mode=lax.GatherScatterMode.PROMISE_IN_BOUNDS)


def _make_sc_half(half):
  @functools.partial(
      pl.kernel,
      out_type=jax.ShapeDtypeStruct((BNH, DROW), jnp.float32),
      mesh=_sc_mesh,
      compiler_params=pltpu.CompilerParams(
          needs_layout_passes=False, use_tc_tiling_on_sc=False
      ),
      scratch_types=[
          pltpu.VMEM((QPW * K,), jnp.int32),
          pltpu.VMEM((N * 4,), jnp.float32),
          pltpu.VMEM((RING, CPK, Din), jnp.float32),
          pltpu.VMEM((CP, DROW), jnp.float32),
      ] + [pltpu.SemaphoreType.DMA] * RING,
  )
  def _sc_flex(feat_hbm, idx_hbm, pos_hbm, out_hbm, idx_v, pos_v,
               rows_v, acc_v, *sems):
    otab = out_hbm
    ftab = feat_hbm
    wid = lax.axis_index("c") * NS + lax.axis_index("s")
    oq = wid * QPW
    bn0 = (oq // N) * N

    pltpu.sync_copy(idx_hbm.at[pl.ds(oq * K, QPW * K)], idx_v)
    pltpu.sync_copy(pos_hbm.at[pl.ds(bn0 * 4, N * 4)], pos_v)

    def fire(g, slot):
        gw = lax.rem(g, NCHUNK)
        pltpu.async_copy(
            ftab.at[idx_v.at[pl.ds(gw * CPK, CPK)]], rows_v.at[slot], sems[slot]
        )

    def wait(g, slot):
        gw = lax.rem(g, NCHUNK)
        pltpu.make_async_copy(
            ftab.at[idx_v.at[pl.ds(gw * CPK, CPK)]], rows_v.at[slot], sems[slot]
        ).wait()

    for j in range(RING - 1):
        fire(j, j)

    def ring_body(gg, carry):
        for s in range(RING):
            g = gg * RING + s
            fire(g + RING - 1, (s + RING - 1) % RING)
            wait(g, s)

            def point_body(p2, c2):
              for dp in range(2):
                p = p2 * 2 + dp
                r0 = p * K
                nb = idx_v[pl.ds(g * CPK + r0, K)] - bn0
                nb4 = nb * 4
                w0 = plsc.load_gather(pos_v, [nb4])
                w1 = plsc.load_gather(pos_v, [nb4 + 1])
                w2 = plsc.load_gather(pos_v, [nb4 + 2])
                z = jnp.zeros((16,), jnp.float32)
                s0a, s0b = z, z
                u0a, u0b, u1a, u1b, u2a, u2b = z, z, z, z, z, z
                for k in range(K):
                    f0 = rows_v[s, r0 + k, pl.ds(0, 16)]
                    f1 = rows_v[s, r0 + k, pl.ds(16, 16)]
                    b0 = _bcast_lane(w0, k)
                    b1 = _bcast_lane(w1, k)
                    b2 = _bcast_lane(w2, k)
                    s0a = s0a + f0
                    s0b = s0b + f1
                    u0a = u0a + b0 * f0
                    u0b = u0b + b0 * f1
                    u1a = u1a + b1 * f0
                    u1b = u1b + b1 * f1
                    u2a = u2a + b2 * f0
                    u2b = u2b + b2 * f1
                acc_v[p, pl.ds(0, 16)] = s0a
                acc_v[p, pl.ds(16, 16)] = s0b
                acc_v[p, pl.ds(32, 16)] = u0a
                acc_v[p, pl.ds(48, 16)] = u0b
                acc_v[p, pl.ds(64, 16)] = u1a
                acc_v[p, pl.ds(80, 16)] = u1b
                acc_v[p, pl.ds(96, 16)] = u2a
                acc_v[p, pl.ds(112, 16)] = u2b
              return c2

            lax.fori_loop(0, CP // 2, point_body, 0)
            pltpu.sync_copy(acc_v, otab.at[pl.ds(oq + g * CP, CP)])
        return carry

    lax.fori_loop(0, NCHUNK // RING, ring_body, 0)
    for j in range(RING - 1):
        wait(NCHUNK + j, (NCHUNK + j) % RING)

  return _sc_flex


_sc_halves = tuple(_make_sc_half(h) for h in range(NHALF))


TN = 2048


def _post_body(t_ref, p_ref, tf_ref, th_ref, bias_ref, fb_ref, o_ref):
    t = t_ref[...]
    s = t[:, 0:Din]
    u = t[:, Din:DROW]
    out = jnp.dot(u, tf_ref[...], preferred_element_type=jnp.float32)
    out = out + jnp.dot(s, bias_ref[...], preferred_element_type=jnp.float32)
    v = jnp.dot(s, th_ref[...], preferred_element_type=jnp.float32)
    for p in range(Dp):
        out = out - p_ref[:, p:p + 1] * v[:, p * Dout:(p + 1) * Dout]
    out = out + fb_ref[...]
    o_ref[...] = jnp.maximum(out, 0.0)


_post = pl.pallas_call(
    _post_body,
    grid=(BNH // TN,),
    in_specs=[
        pl.BlockSpec((TN, DROW), lambda i: (i, 0)),
        pl.BlockSpec((TN, 4), lambda i: (i, 0)),
        pl.BlockSpec((Dp * Din, Dout), lambda i: (0, 0)),
        pl.BlockSpec((Din, Dp * Dout), lambda i: (0, 0)),
        pl.BlockSpec((Din, Dout), lambda i: (0, 0)),
        pl.BlockSpec((1, Dout), lambda i: (0, 0)),
    ],
    out_specs=pl.BlockSpec((TN, Dout), lambda i: (i, 0)),
    out_shape=jax.ShapeDtypeStruct((BNH, Dout), jnp.float32),
)


@jax.jit
def kernel(features, positions, neighborhoods, position_theta, position_bias, feature_bias):
    feat_flat = features.transpose(0, 2, 1).reshape(BN, Din)
    posT = positions.transpose(0, 2, 1).reshape(BN, Dp)
    posT4 = jnp.concatenate([posT, jnp.zeros((BN, 1), jnp.float32)], axis=1)
    pos_flat = posT4.reshape(BN * 4)
    offs = (jnp.arange(B, dtype=jnp.int32) * N)[:, None, None]
    idx = (neighborhoods + offs).transpose(0, 2, 1).reshape(BN * K)

    theta_flat = position_theta.reshape(Dp * Din, Dout)
    theta_h = jnp.concatenate([position_theta[p] for p in range(Dp)], axis=1)
    fb = feature_bias.reshape(1, Dout)

    outs = []
    for h in range(NHALF):
        feat_h = feat_flat[h * BNH:(h + 1) * BNH]
        idx_h = idx[h * BNH * K:(h + 1) * BNH * K] - h * BNH
        pos_h = pos_flat[h * BNH * 4:(h + 1) * BNH * 4]
        t_h = _sc_halves[h](feat_h, idx_h, pos_h)
        p_h = posT4[h * BNH:(h + 1) * BNH]
        o_h = _post(t_h, p_h, theta_flat, theta_h, position_bias, fb)
        outs.append(o_h.reshape(B // NHALF, N, Dout).transpose(0, 2, 1))
    return jnp.concatenate(outs, axis=0)


# speedup vs baseline: 1.0394x; 1.0394x over previous
import functools

import jax
import jax.numpy as jnp
from jax import lax
from jax.experimental import pallas as pl
from jax.experimental.pallas import tpu as pltpu
from jax.experimental.pallas import tpu_sc as plsc

B, Din, Dp, K, N, Dout = 4, 32, 3, 16, 16384, 32
BN = B * N
DROW = (1 + Dp) * Din

NC, NS = 2, 16
NW = NC * NS
NHALF = 2
BNH = BN // NHALF
QPW = BNH // NW
CP = 8
CPK = CP * K
NCHUNK = QPW // CP
RING = 4

_sc_mesh = plsc.VectorSubcoreMesh(core_axis_name="c", subcore_axis_name="s")

_gdn = lax.GatherDimensionNumbers(
    offset_dims=(), collapsed_slice_dims=(0,), start_index_map=(0,)
)


def _bcast_lane(v, k):
    idx = jnp.full((K, 1), k, jnp.int32)
    return lax.gather(v, idx, _gdn, slice_sizes=(1,),
                      mode=lax.GatherScatterMode.PROMISE_IN_BOUNDS)


def _make_sc_half(half):
  @functools.partial(
      pl.kernel,
      out_type=jax.ShapeDtypeStruct((BNH, DROW), jnp.float32),
      mesh=_sc_mesh,
      compiler_params=pltpu.CompilerParams(
          needs_layout_passes=False, use_tc_tiling_on_sc=False
      ),
      scratch_types=[
          pltpu.VMEM((QPW * K,), jnp.int32),
          pltpu.VMEM((N * 4,), jnp.float32),
          pltpu.VMEM((RING, CPK, Din), jnp.float32),
          pltpu.VMEM((CP, DROW), jnp.float32),
      ] + [pltpu.SemaphoreType.DMA] * RING,
  )
  def _sc_flex(feat_hbm, idx_hbm, pos_hbm, out_hbm, idx_v, pos_v,
               rows_v, acc_v, *sems):
    otab = out_hbm
    ftab = feat_hbm
    wid = lax.axis_index("c") * NS + lax.axis_index("s")
    oq = wid * QPW
    bn0 = (oq // N) * N

    pltpu.sync_copy(idx_hbm.at[pl.ds(oq * K, QPW * K)], idx_v)
    pltpu.sync_copy(pos_hbm.at[pl.ds(bn0 * 4, N * 4)], pos_v)

    def fire(g, slot):
        gw = lax.rem(g, NCHUNK)
        pltpu.async_copy(
            ftab.at[idx_v.at[pl.ds(gw * CPK, CPK)]], rows_v.at[slot], sems[slot]
        )

    def wait(g, slot):
        gw = lax.rem(g, NCHUNK)
        pltpu.make_async_copy(
            ftab.at[idx_v.at[pl.ds(gw * CPK, CPK)]], rows_v.at[slot], sems[slot]
        ).wait()

    for j in range(RING - 1):
        fire(j, j)

    def ring_body(gg, carry):
        for s in range(RING):
            g = gg * RING + s
            fire(g + RING - 1, (s + RING - 1) % RING)
            wait(g, s)

            def point_body(p, c2):
                r0 = p * K
                nb = idx_v[pl.ds(g * CPK + r0, K)] - bn0
                nb4 = nb * 4
                w0 = plsc.load_gather(pos_v, [nb4])
                w1 = plsc.load_gather(pos_v, [nb4 + 1])
                w2 = plsc.load_gather(pos_v, [nb4 + 2])
                z = jnp.zeros((16,), jnp.float32)
                s0a, s0b = z, z
                u0a, u0b, u1a, u1b, u2a, u2b = z, z, z, z, z, z
                for k in range(K):
                    f0 = rows_v[s, r0 + k, pl.ds(0, 16)]
                    f1 = rows_v[s, r0 + k, pl.ds(16, 16)]
                    b0 = _bcast_lane(w0, k)
                    b1 = _bcast_lane(w1, k)
                    b2 = _bcast_lane(w2, k)
                    s0a = s0a + f0
                    s0b = s0b + f1
                    u0a = u0a + b0 * f0
                    u0b = u0b + b0 * f1
                    u1a = u1a + b1 * f0
                    u1b = u1b + b1 * f1
                    u2a = u2a + b2 * f0
                    u2b = u2b + b2 * f1
                acc_v[p, pl.ds(0, 16)] = s0a
                acc_v[p, pl.ds(16, 16)] = s0b
                acc_v[p, pl.ds(32, 16)] = u0a
                acc_v[p, pl.ds(48, 16)] = u0b
                acc_v[p, pl.ds(64, 16)] = u1a
                acc_v[p, pl.ds(80, 16)] = u1b
                acc_v[p, pl.ds(96, 16)] = u2a
                acc_v[p, pl.ds(112, 16)] = u2b
                return c2

            lax.fori_loop(0, CP, point_body, 0)
            pltpu.sync_copy(acc_v, otab.at[pl.ds(oq + g * CP, CP)])
        return carry

    lax.fori_loop(0, NCHUNK // RING, ring_body, 0)
    for j in range(RING - 1):
        wait(NCHUNK + j, (NCHUNK + j) % RING)

  return _sc_flex


_sc_halves = tuple(_make_sc_half(h) for h in range(NHALF))


TN = 2048


def _post_body(t_ref, p_ref, tf_ref, th_ref, bias_ref, fb_ref, o_ref):
    t = t_ref[...]
    s = t[:, 0:Din]
    u = t[:, Din:DROW]
    out = jnp.dot(u, tf_ref[...], preferred_element_type=jnp.float32)
    out = out + jnp.dot(s, bias_ref[...], preferred_element_type=jnp.float32)
    v = jnp.dot(s, th_ref[...], preferred_element_type=jnp.float32)
    for p in range(Dp):
        out = out - p_ref[:, p:p + 1] * v[:, p * Dout:(p + 1) * Dout]
    out = out + fb_ref[...]
    o_ref[...] = jnp.maximum(out, 0.0)


_post = pl.pallas_call(
    _post_body,
    grid=(BNH // TN,),
    in_specs=[
        pl.BlockSpec((TN, DROW), lambda i: (i, 0)),
        pl.BlockSpec((TN, 4), lambda i: (i, 0)),
        pl.BlockSpec((Dp * Din, Dout), lambda i: (0, 0)),
        pl.BlockSpec((Din, Dp * Dout), lambda i: (0, 0)),
        pl.BlockSpec((Din, Dout), lambda i: (0, 0)),
        pl.BlockSpec((1, Dout), lambda i: (0, 0)),
    ],
    out_specs=pl.BlockSpec((TN, Dout), lambda i: (i, 0)),
    out_shape=jax.ShapeDtypeStruct((BNH, Dout), jnp.float32),
)


@jax.jit
def kernel(features, positions, neighborhoods, position_theta, position_bias, feature_bias):
    feat_flat = features.transpose(0, 2, 1).reshape(BN, Din)
    posT = positions.transpose(0, 2, 1).reshape(BN, Dp)
    posT4 = jnp.concatenate([posT, jnp.zeros((BN, 1), jnp.float32)], axis=1)
    pos_flat = posT4.reshape(BN * 4)
    offs = (jnp.arange(B, dtype=jnp.int32) * N)[:, None, None]
    idx = (neighborhoods + offs).transpose(0, 2, 1).reshape(BN * K)

    theta_flat = position_theta.reshape(Dp * Din, Dout)
    theta_h = jnp.concatenate([position_theta[p] for p in range(Dp)], axis=1)
    fb = feature_bias.reshape(1, Dout)

    outs = []
    for h in range(NHALF):
        feat_h = feat_flat[h * BNH:(h + 1) * BNH]
        idx_h = idx[h * BNH * K:(h + 1) * BNH * K] - h * BNH
        pos_h = pos_flat[h * BNH * 4:(h + 1) * BNH * 4]
        t_h = _sc_halves[h](feat_h, idx_h, pos_h)
        p_h = posT4[h * BNH:(h + 1) * BNH]
        o_h = _post(t_h, p_h, theta_flat, theta_h, position_bias, fb)
        outs.append(o_h.reshape(B // NHALF, N, Dout).transpose(0, 2, 1))
    return jnp.concatenate(outs, axis=0)
